# baseline (speedup 1.0000x reference)
import functools
import jax
import jax.numpy as jnp
from jax import lax
from jax.experimental import pallas as pl
from jax.experimental.pallas import tpu as pltpu
from jax.experimental.pallas import tpu_sc as plsc

_INPUT = 2048
_OUT = 1024
_NBITS = 6
_BATCH = 512
_NLUT = 1 << _NBITS
_NW = 32
_BPW = _BATCH // _NW
_L = 16
_NCHUNK = _OUT // _L


def _lut_body(x_hbm, map_hbm, luts_hbm, out_hbm, x_v, map_v, luts_v, out_v):
    wid = lax.axis_index("s") * 2 + lax.axis_index("c")
    base = wid * _BPW
    for b in range(_BPW):
        pltpu.sync_copy(x_hbm.at[base + b], x_v.at[pl.ds(b * _INPUT, _INPUT)])
    pltpu.sync_copy(map_hbm, map_v)
    pltpu.sync_copy(luts_hbm, luts_v)

    jiota = lax.iota(jnp.int32, _L)

    @plsc.parallel_loop(0, _NCHUNK)
    def jc_body(jc):
        jb = jc * _L
        junits = jiota + jb
        lbase = junits * _NLUT
        mbase = junits * _NBITS
        idxs = [plsc.load_gather(map_v, [mbase + k]) for k in range(_NBITS)]
        for b in range(_BPW):
            addr = jnp.zeros((_L,), jnp.int32)
            for k in range(_NBITS):
                g = plsc.load_gather(x_v, [idxs[k] + (b * _INPUT)])
                addr = addr + jnp.where(g > 0.0, jnp.int32(1 << k), jnp.int32(0))
            lv = plsc.load_gather(luts_v, [lbase + addr])
            out_v[pl.ds(b * _OUT + jb, _L)] = jnp.where(lv > 0.0, 1.0, 0.0)

    for b in range(_BPW):
        pltpu.sync_copy(out_v.at[pl.ds(b * _OUT, _OUT)], out_hbm.at[base + b])


@jax.jit
def _lut_forward(x, mapping_flat, luts_flat):
    mesh = plsc.VectorSubcoreMesh(core_axis_name="c", subcore_axis_name="s")
    fn = functools.partial(
        pl.kernel,
        mesh=mesh,
        compiler_params=pltpu.CompilerParams(needs_layout_passes=False),
        out_type=jax.ShapeDtypeStruct((_BATCH, _OUT), jnp.float32),
        scratch_types=[
            pltpu.VMEM((_BPW * _INPUT,), jnp.float32),
            pltpu.VMEM((_OUT * _NBITS,), jnp.int32),
            pltpu.VMEM((_OUT * _NLUT,), jnp.float32),
            pltpu.VMEM((_BPW * _OUT,), jnp.float32),
        ],
    )(_lut_body)
    return fn(x, mapping_flat, luts_flat)


def kernel(x, mapping, luts):
    return _lut_forward(x, mapping.reshape(-1), luts.reshape(-1))
